# Initial kernel scaffold; baseline (speedup 1.0000x reference)
import jax
import jax.numpy as jnp
from jax.experimental import pallas as pl


def kernel(state, W1, b1, W2, b2, W3, b3, gamma, beta):
    raise NotImplementedError("write your pallas kernel here")


# speedup vs baseline: 1.9250x; 1.9250x over previous
import jax
import jax.numpy as jnp
from jax.experimental import pallas as pl

_B = 16384
_IN = 512
_H = 256
_LATENT = 128
_TILE = 1024


def _fused_mlp_ln(x_ref, w1_ref, b1_ref, w2_ref, b2_ref, w3_ref, b3_ref,
                  g_ref, bt_ref, o_ref):
    x = x_ref[...]
    h = jax.lax.dot(x, w1_ref[...], preferred_element_type=jnp.float32)
    h = jnp.maximum(h + b1_ref[...], 0.0)
    h = jax.lax.dot(h, w2_ref[...], preferred_element_type=jnp.float32)
    h = jnp.maximum(h + b2_ref[...], 0.0)
    h = jax.lax.dot(h, w3_ref[...], preferred_element_type=jnp.float32)
    h = h + b3_ref[...]
    mean = jnp.mean(h, axis=-1, keepdims=True)
    hc = h - mean
    var = jnp.mean(hc * hc, axis=-1, keepdims=True)
    hn = hc * jax.lax.rsqrt(var + 1e-5)
    o_ref[...] = hn * g_ref[...] + bt_ref[...]


def kernel(state, W1, b1, W2, b2, W3, b3, gamma, beta):
    b1r = b1.reshape(1, _H)
    b2r = b2.reshape(1, _H)
    b3r = b3.reshape(1, _LATENT)
    gr = gamma.reshape(1, _LATENT)
    btr = beta.reshape(1, _LATENT)
    grid = (_B // _TILE,)
    return pl.pallas_call(
        _fused_mlp_ln,
        grid=grid,
        in_specs=[
            pl.BlockSpec((_TILE, _IN), lambda i: (i, 0)),
            pl.BlockSpec((_IN, _H), lambda i: (0, 0)),
            pl.BlockSpec((1, _H), lambda i: (0, 0)),
            pl.BlockSpec((_H, _H), lambda i: (0, 0)),
            pl.BlockSpec((1, _H), lambda i: (0, 0)),
            pl.BlockSpec((_H, _LATENT), lambda i: (0, 0)),
            pl.BlockSpec((1, _LATENT), lambda i: (0, 0)),
            pl.BlockSpec((1, _LATENT), lambda i: (0, 0)),
            pl.BlockSpec((1, _LATENT), lambda i: (0, 0)),
        ],
        out_specs=pl.BlockSpec((_TILE, _LATENT), lambda i: (i, 0)),
        out_shape=jax.ShapeDtypeStruct((_B, _LATENT), jnp.float32),
    )(state, W1, b1r, W2, b2r, W3, b3r, gr, btr)
